# Initial kernel scaffold; baseline (speedup 1.0000x reference)
import jax
import jax.numpy as jnp
from jax.experimental import pallas as pl


def kernel(hidden_states, router_indices, routing_weights, swiglu_limit, gate_up_proj_blocks, gate_up_proj_scales, gate_up_proj_bias, down_proj_blocks, down_proj_scales, down_proj_bias):
    raise NotImplementedError("write your pallas kernel here")


# speedup vs baseline: 4.0926x; 4.0926x over previous
import numpy as np
import jax
import jax.numpy as jnp
from jax.experimental import pallas as pl
from jax.experimental.pallas import tpu as pltpu

_NE = 8
_D = 1024
_T = 4096
_CB = 256


def _build_perms():
    p = np.arange(_D)
    b = (p >= _D // 2).astype(np.int64)
    m = p - (_D // 2) * b
    kperm = 32 * (m // 16) + 2 * (m % 16) + b
    rowperm = np.concatenate([2 * kperm, 2 * kperm + 1])
    return kperm, rowperm


_KPERM, _ROWPERM = _build_perms()


def _decode_fp4(nib, scale):
    mag = nib & 7
    mf = mag.astype(jnp.float32)
    dec = jnp.where(mag < 5, mf * 0.5, mf - 2.0)
    dec = jnp.where(mag == 7, 6.0, dec)
    sgn = jnp.where((nib & 8) == 0, 1.0, -1.0)
    return dec * sgn * scale


def _decode_body(gub, gus, dnb, dns, gw_out, dw_out):
    def dec(bref, sref, oref):
        bv = bref[0].astype(jnp.int32)
        sv = sref[0].astype(jnp.int32)
        scale = jax.lax.bitcast_convert_type(sv << 23, jnp.float32)
        oref[0, :, : _D // 2] = _decode_fp4(bv & 15, scale).astype(jnp.bfloat16)
        oref[0, :, _D // 2:] = _decode_fp4(bv >> 4, scale).astype(jnp.bfloat16)

    dec(gub, gus, gw_out)
    dec(dnb, dns, dw_out)


def _decode_weights(gub, gus, dnb, dns):
    return pl.pallas_call(
        _decode_body,
        grid=(_NE,),
        in_specs=[
            pl.BlockSpec((1, 2 * _D, _D // 2), lambda e: (e, 0, 0)),
            pl.BlockSpec((1, 2 * _D, _D // 2), lambda e: (e, 0, 0)),
            pl.BlockSpec((1, _D, _D // 2), lambda e: (e, 0, 0)),
            pl.BlockSpec((1, _D, _D // 2), lambda e: (e, 0, 0)),
        ],
        out_specs=[
            pl.BlockSpec((1, 2 * _D, _D), lambda e: (e, 0, 0)),
            pl.BlockSpec((1, _D, _D), lambda e: (e, 0, 0)),
        ],
        out_shape=[
            jax.ShapeDtypeStruct((_NE, 2 * _D, _D), jnp.bfloat16),
            jax.ShapeDtypeStruct((_NE, _D, _D), jnp.bfloat16),
        ],
    )(gub, gus, dnb, dns)


def _ffn_body(lim, xq, gw, b1, dw, b2, weff, out):
    e = pl.program_id(0)

    @pl.when(e == 0)
    def _():
        out[...] = jnp.zeros_like(out)

    limv = lim[0, 0]

    def chunk(c, carry):
        x = xq[pl.ds(c * _CB, _CB), :]
        gu = jax.lax.dot_general(
            x, gw[0], (((1,), (1,)), ((), ())),
            preferred_element_type=jnp.float32)
        gu = gu + b1[0]
        g = jnp.minimum(gu[:, :_D], limv)
        l = jnp.clip(gu[:, _D:], -limv, limv)
        act = (g * jax.nn.sigmoid(1.702 * g) * (l + 1.0)).astype(jnp.bfloat16)
        y = jax.lax.dot_general(
            act, dw[0], (((1,), (1,)), ((), ())),
            preferred_element_type=jnp.float32)
        y = y + b2[0]
        w = weff[0, 0, pl.ds(c * _CB, _CB)]
        out[pl.ds(c * _CB, _CB), :] += y * w[:, None]
        return carry

    jax.lax.fori_loop(0, _T // _CB, chunk, 0)


def _ffn(lim, xq, gw, b1, dw, b2, weff):
    return pl.pallas_call(
        _ffn_body,
        grid=(_NE,),
        in_specs=[
            pl.BlockSpec(memory_space=pltpu.SMEM),
            pl.BlockSpec((_T, _D), lambda e: (0, 0)),
            pl.BlockSpec((1, 2 * _D, _D), lambda e: (e, 0, 0)),
            pl.BlockSpec((1, 1, 2 * _D), lambda e: (e, 0, 0)),
            pl.BlockSpec((1, _D, _D), lambda e: (e, 0, 0)),
            pl.BlockSpec((1, 1, _D), lambda e: (e, 0, 0)),
            pl.BlockSpec((1, 1, _T), lambda e: (e, 0, 0)),
        ],
        out_specs=pl.BlockSpec((_T, _D), lambda e: (0, 0)),
        out_shape=jax.ShapeDtypeStruct((_T, _D), jnp.float32),
    )(lim, xq, gw, b1, dw, b2, weff)


def kernel(hidden_states, router_indices, routing_weights, swiglu_limit,
           gate_up_proj_blocks, gate_up_proj_scales, gate_up_proj_bias,
           down_proj_blocks, down_proj_scales, down_proj_bias):
    flat = hidden_states.reshape(-1, _D)
    xq = flat[:, _KPERM].astype(jnp.bfloat16)

    gub = gate_up_proj_blocks.reshape(_NE, 2 * _D, _D // 2)[:, _ROWPERM]
    gus = jnp.repeat(gate_up_proj_scales[:, _ROWPERM], 16, axis=-1)
    dnb = down_proj_blocks.reshape(_NE, _D, _D // 2)
    dns = jnp.repeat(down_proj_scales, 16, axis=-1)
    b1p = gate_up_proj_bias[:, _ROWPERM].reshape(_NE, 1, 2 * _D)
    b2r = down_proj_bias.reshape(_NE, 1, _D)

    cnt = (router_indices[:, :, None] == jnp.arange(_NE)[None, None, :]
           ).astype(jnp.float32).sum(axis=1)
    weff = (routing_weights * cnt).T.reshape(_NE, 1, _T)

    lim = jnp.full((1, 1), swiglu_limit, jnp.float32)

    gw, dw = _decode_weights(gub, gus, dnb, dns)
    out = _ffn(lim, xq, gw, b1p, dw, b2r, weff)
    return out.reshape(*hidden_states.shape[:-1], _D).astype(hidden_states.dtype)
